# Initial kernel scaffold; baseline (speedup 1.0000x reference)
import jax
import jax.numpy as jnp
from jax.experimental import pallas as pl


def kernel(neighbors_l0, neighbors_l1, neighbors_l2, offsets_l1, offsets_l2, user_feature_indices, user_feature_offsets, item_feature_indices, item_feature_offsets, user_feature_emb, item_feature_emb, user_proj_W, user_proj_b, item_proj_W, item_proj_b, w0_W, w0_b, w1_W, w1_b):
    raise NotImplementedError("write your pallas kernel here")


# speedup vs baseline: 178.8340x; 178.8340x over previous
import functools

import jax
import jax.numpy as jnp
from jax import lax
from jax.experimental import pallas as pl
from jax.experimental.pallas import tpu as pltpu
from jax.experimental.pallas import tpu_sc as plsc

D = 128
N_USERS = 50000
N_ITEMS = 50000
B = 1024
FANOUT = 16
FEAT = 8

NC, NS = 2, 16
NW = NC * NS

U_PAD = 50176
PU = U_PAD // NW
CU = 16
NCH_A = PU // CU

E2 = B * FANOUT * FANOUT
E2W = E2 // NW
CE_B = 128
NCH_B = E2W // CE_B

E1 = B * FANOUT
E1W = E1 // NW
CE_C = 16
NCH_C = E1W // CE_C

RMID = 2048


def _worker_id():
    return lax.axis_index("s") * NC + lax.axis_index("c")


def _mean_rows(rows_v, acc_v, n_out, group, scale):
    def per_g(g, carry):
        for dd in range(D // 16):
            sl = pl.ds(dd * 16, 16)
            acc = rows_v[g * group, sl]
            for f in range(1, group):
                acc = acc + rows_v[g * group + f, sl]
            acc_v[g, sl] = acc * scale
        return carry
    lax.fori_loop(0, n_out, per_g, 0)


@functools.lru_cache(maxsize=None)
def _build_sc_kernels():
    mesh = plsc.VectorSubcoreMesh(core_axis_name="c", subcore_axis_name="s")

    @functools.partial(
        pl.kernel, mesh=mesh,
        out_type=jax.ShapeDtypeStruct((U_PAD, D), jnp.float32),
        scratch_types=[
            pltpu.VMEM((PU * FEAT,), jnp.int32),
            pltpu.VMEM((CU * FEAT, D), jnp.float32),
            pltpu.VMEM((CU, D), jnp.float32),
            pltpu.SemaphoreType.DMA,
        ],
    )
    def ufm_kernel(idx_hbm, emb_hbm, out_hbm, idx_v, rows_v, acc_v, sem):
        wid = _worker_id()
        ub = wid * PU
        pltpu.sync_copy(idx_hbm.at[pl.ds(ub * FEAT, PU * FEAT)], idx_v)

        def chunk(i, carry):
            pltpu.async_copy(
                emb_hbm.at[idx_v.at[pl.ds(i * CU * FEAT, CU * FEAT)]],
                rows_v, sem).wait()
            _mean_rows(rows_v, acc_v, CU, FEAT, 1.0 / FEAT)
            pltpu.sync_copy(acc_v, out_hbm.at[pl.ds(ub + i * CU, CU)])
            return carry
        lax.fori_loop(0, NCH_A, chunk, 0)

    @functools.partial(
        pl.kernel, mesh=mesh,
        out_type=(jax.ShapeDtypeStruct((B, D), jnp.float32),
                  jax.ShapeDtypeStruct((E1, D), jnp.float32)),
        scratch_types=[
            pltpu.VMEM((E2W,), jnp.int32),
            pltpu.VMEM((CE_B, D), jnp.float32),
            pltpu.VMEM((CE_B // FANOUT, D), jnp.float32),
            pltpu.VMEM((B // NW,), jnp.int32),
            pltpu.VMEM((B // NW, D), jnp.float32),
            pltpu.SemaphoreType.DMA,
        ],
    )
    def l2_kernel(ufm_hbm, n0_hbm, n2_hbm, h0_hbm, m2_hbm,
                  idx_v, rows_v, acc_v, nbr0_v, rows0_v, sem):
        wid = _worker_id()
        r0 = wid * (B // NW)
        pltpu.sync_copy(n0_hbm.at[pl.ds(r0, B // NW)], nbr0_v)
        pltpu.async_copy(ufm_hbm.at[nbr0_v], rows0_v, sem).wait()
        pltpu.sync_copy(rows0_v, h0_hbm.at[pl.ds(r0, B // NW)])
        eb = wid * E2W
        gb = wid * (E2W // FANOUT)
        pltpu.sync_copy(n2_hbm.at[pl.ds(eb, E2W)], idx_v)

        def chunk(i, carry):
            pltpu.async_copy(
                ufm_hbm.at[idx_v.at[pl.ds(i * CE_B, CE_B)]],
                rows_v, sem).wait()
            _mean_rows(rows_v, acc_v, CE_B // FANOUT, FANOUT, 1.0 / FANOUT)
            pltpu.sync_copy(
                acc_v, m2_hbm.at[pl.ds(gb + i * (CE_B // FANOUT),
                                       CE_B // FANOUT)])
            return carry
        lax.fori_loop(0, NCH_B, chunk, 0)

    @functools.partial(
        pl.kernel, mesh=mesh,
        out_type=(jax.ShapeDtypeStruct((E1, D), jnp.float32),
                  jax.ShapeDtypeStruct((B, D), jnp.float32)),
        scratch_types=[
            pltpu.VMEM((E1W * FEAT,), jnp.int32),
            pltpu.VMEM((CE_C * FEAT,), jnp.int32),
            pltpu.VMEM((CE_C * FEAT, D), jnp.float32),
            pltpu.VMEM((CE_C, D), jnp.float32),
            pltpu.VMEM((1, D), jnp.float32),
            pltpu.SemaphoreType.DMA,
        ],
    )
    def item_kernel(flat_hbm, ifi_hbm, emb_hbm, h1_hbm, m1_hbm,
                    fidx_v, idx8_v, rows_v, h1acc_v, m1acc_v, sem):
        wid = _worker_id()
        eb = wid * E1W
        pltpu.sync_copy(flat_hbm.at[pl.ds(eb * FEAT, E1W * FEAT)], fidx_v)

        def chunk(i, carry):
            pltpu.async_copy(
                ifi_hbm.at[fidx_v.at[pl.ds(i * CE_C * FEAT, CE_C * FEAT)]],
                idx8_v, sem).wait()
            pltpu.async_copy(emb_hbm.at[idx8_v], rows_v, sem).wait()
            _mean_rows(rows_v, h1acc_v, CE_C, FEAT, 1.0 / FEAT)
            pltpu.sync_copy(h1acc_v, h1_hbm.at[pl.ds(eb + i * CE_C, CE_C)])
            _mean_rows(h1acc_v, m1acc_v, 1, FANOUT, 1.0 / FANOUT)
            pltpu.sync_copy(m1acc_v, m1_hbm.at[pl.ds(wid * NCH_C + i, 1)])
            return carry
        lax.fori_loop(0, NCH_C, chunk, 0)

    return ufm_kernel, l2_kernel, item_kernel


def _mid_body(h1r, m2r, Wi, bi, Wu, bu, W0a, W0b, b0, P, mm1):
    h1 = jnp.dot(h1r[...], Wi[...], preferred_element_type=jnp.float32) + bi[...]
    m2 = jnp.dot(m2r[...], Wu[...], preferred_element_type=jnp.float32) + bu[...]
    nh1 = jnp.maximum(
        jnp.dot(h1, W0a[...], preferred_element_type=jnp.float32)
        + jnp.dot(m2, W0b[...], preferred_element_type=jnp.float32)
        + b0[...], 0.0)
    mm1[...] = jnp.dot(P[...], nh1, preferred_element_type=jnp.float32)


def _head_body(h0r, m1r, mm1, Wu, bu, Wi, bi, W0a, W0b, b0, W1a, W1b, b1, out):
    h0 = jnp.dot(h0r[...], Wu[...], preferred_element_type=jnp.float32) + bu[...]
    m1 = jnp.dot(m1r[...], Wi[...], preferred_element_type=jnp.float32) + bi[...]
    nh0 = jnp.maximum(
        jnp.dot(h0, W0a[...], preferred_element_type=jnp.float32)
        + jnp.dot(m1, W0b[...], preferred_element_type=jnp.float32)
        + b0[...], 0.0)
    out[...] = (jnp.dot(nh0, W1a[...], preferred_element_type=jnp.float32)
                + jnp.dot(mm1[...], W1b[...], preferred_element_type=jnp.float32)
                + b1[...])


def kernel(neighbors_l0, neighbors_l1, neighbors_l2, offsets_l1, offsets_l2,
           user_feature_indices, user_feature_offsets, item_feature_indices,
           item_feature_offsets, user_feature_emb, item_feature_emb,
           user_proj_W, user_proj_b, item_proj_W, item_proj_b,
           w0_W, w0_b, w1_W, w1_b):
    n0 = neighbors_l0.astype(jnp.int32)
    n1 = neighbors_l1.astype(jnp.int32)
    n2 = neighbors_l2.astype(jnp.int32)
    ufi = user_feature_indices.astype(jnp.int32)
    ifi = item_feature_indices.astype(jnp.int32)

    ufi_pad = jnp.pad(ufi, (0, (U_PAD - N_USERS) * FEAT))
    flat_item = (n1[:, None] * FEAT
                 + jnp.arange(FEAT, dtype=jnp.int32)).reshape(-1)

    ufm_kernel, l2_kernel, item_kernel = _build_sc_kernels()
    ufm = ufm_kernel(ufi_pad, user_feature_emb)
    h0_raw, m2_raw = l2_kernel(ufm, n0, n2)
    h1_raw, m1_raw = item_kernel(flat_item, ifi, item_feature_emb)

    W0a, W0b = w0_W[:D], w0_W[D:]
    W1a, W1b = w1_W[:D], w1_W[D:]
    bu2, bi2 = user_proj_b[None, :], item_proj_b[None, :]
    b02, b12 = w0_b[None, :], w1_b[None, :]
    pool = jnp.kron(jnp.eye(RMID // FANOUT, dtype=jnp.float32),
                    jnp.full((1, FANOUT), 1.0 / FANOUT, dtype=jnp.float32))

    full = lambda s: pl.BlockSpec(s, lambda i: (0, 0))
    mm1 = pl.pallas_call(
        _mid_body,
        grid=(E1 // RMID,),
        in_specs=[
            pl.BlockSpec((RMID, D), lambda i: (i, 0)),
            pl.BlockSpec((RMID, D), lambda i: (i, 0)),
            full((D, D)), full((1, D)), full((D, D)), full((1, D)),
            full((D, D)), full((D, D)), full((1, D)),
            full((RMID // FANOUT, RMID)),
        ],
        out_specs=pl.BlockSpec((RMID // FANOUT, D), lambda i: (i, 0)),
        out_shape=jax.ShapeDtypeStruct((B, D), jnp.float32),
    )(h1_raw, m2_raw, item_proj_W, bi2, user_proj_W, bu2, W0a, W0b, b02, pool)

    out = pl.pallas_call(
        _head_body,
        out_shape=jax.ShapeDtypeStruct((B, D), jnp.float32),
    )(h0_raw, m1_raw, mm1, user_proj_W, bu2, item_proj_W, bi2,
      W0a, W0b, b02, W1a, W1b, b12)
    return out
